# baseline (speedup 1.0000x reference)
import functools
import math

import jax
import jax.numpy as jnp
from jax import lax
from jax.experimental import pallas as pl
from jax.experimental.pallas import tpu as pltpu
from jax.experimental.pallas import tpu_sc as plsc

H = 128
NUM_RBF = 32
N_NODES = 10000
N_EDGES = 320000

NP = 10240


def _node_body(x_ref, vecf_ref, w1_ref, b1_ref, w2_ref, b2_ref, g_ref, b_ref,
               out_ref, *, block_rows):
    x = x_ref[...]
    mu = jnp.mean(x, axis=-1, keepdims=True)
    var = jnp.mean((x - mu) ** 2, axis=-1, keepdims=True)
    xn = (x - mu) * lax.rsqrt(var + 1e-5) * g_ref[...] + b_ref[...]
    t = jnp.dot(xn, w1_ref[...], preferred_element_type=jnp.float32) + b1_ref[...]
    t = jax.nn.silu(t) * (1.0 / 0.6)
    xh = jnp.dot(t, w2_ref[...], preferred_element_type=jnp.float32) + b2_ref[...]
    row = pl.program_id(0) * block_rows + lax.broadcasted_iota(
        jnp.int32, (block_rows, 1), 0)
    xh = jnp.where(row < N_NODES, xh, 0.0)
    out_ref[:, :3 * H] = xh
    out_ref[:, 3 * H:] = jnp.where(row < N_NODES, vecf_ref[...], 0.0)


def _make_nodetab(xp, vecfp):
    blk = 1024
    grid = NP // blk
    return pl.pallas_call(
        functools.partial(_node_body, block_rows=blk),
        grid=(grid,),
        in_specs=[
            pl.BlockSpec((blk, H), lambda i: (i, 0)),
            pl.BlockSpec((blk, 3 * H), lambda i: (i, 0)),
            pl.BlockSpec((H, H), lambda i: (0, 0)),
            pl.BlockSpec((H,), lambda i: (0,)),
            pl.BlockSpec((H, 3 * H), lambda i: (0, 0)),
            pl.BlockSpec((3 * H,), lambda i: (0,)),
            pl.BlockSpec((H,), lambda i: (0,)),
            pl.BlockSpec((H,), lambda i: (0,)),
        ],
        out_specs=pl.BlockSpec((blk, 6 * H), lambda i: (i, 0)),
        out_shape=jax.ShapeDtypeStruct((NP, 6 * H), jnp.float32),
    )


def _edge_body(rbf_ref, w_ref, b_ref, evp_ref, out_ref):
    blk = rbf_ref.shape[0]
    out_ref[:, :3 * H] = jnp.dot(rbf_ref[...], w_ref[...],
                                 preferred_element_type=jnp.float32) + b_ref[...]
    out_ref[:, 3 * H:3 * H + 16] = evp_ref[...]
    out_ref[:, 3 * H + 16:] = jnp.zeros((blk, 512 - 3 * H - 16), jnp.float32)


def _make_edgetab():
    blk = 3200
    grid = N_EDGES // blk
    return pl.pallas_call(
        _edge_body,
        grid=(grid,),
        in_specs=[
            pl.BlockSpec((blk, NUM_RBF), lambda i: (i, 0)),
            pl.BlockSpec((NUM_RBF, 3 * H), lambda i: (0, 0)),
            pl.BlockSpec((3 * H,), lambda i: (0,)),
            pl.BlockSpec((blk, 16), lambda i: (i, 0)),
        ],
        out_specs=pl.BlockSpec((blk, 512), lambda i: (i, 0)),
        out_shape=jax.ShapeDtypeStruct((N_EDGES, 512), jnp.float32),
    )


def kernel(x, vec, edge_index, edge_rbf, edge_vector, W1, b1, W2, b2, Wrbf,
           brbf, ln_g, ln_b):
    inv3 = 1.0 / math.sqrt(3.0)
    invh = 1.0 / math.sqrt(float(H))
    s = jnp.concatenate([
        jnp.full((H,), 1.0, jnp.float32),
        jnp.full((H,), inv3 * invh, jnp.float32),
        jnp.full((H,), invh, jnp.float32),
    ])
    Wrbf_s = Wrbf * s[None, :]
    brbf_s = brbf * s

    xp = jnp.pad(x, ((0, NP - N_NODES), (0, 0)))
    vecfp = jnp.pad(vec.reshape(N_NODES, 3 * H), ((0, NP - N_NODES), (0, 0)))
    evp = jnp.pad(edge_vector, ((0, 0), (0, 13)))

    nodetab = _make_nodetab(xp, vecfp)(
        xp, vecfp, W1, b1, W2, b2, ln_g, ln_b)
    edgetab = _make_edgetab()(edge_rbf, Wrbf_s, brbf_s, evp)

    src = edge_index[0].astype(jnp.int32)
    dst = edge_index[1].astype(jnp.int32)

    zrows = jnp.zeros((ROWS, 512), jnp.float32)
    Y = _sc_edge_kernel()(nodetab, edgetab, dst, src, zrows)

    dx = Y[:N_NODES, :H]
    dvec = Y[:N_NODES, H:].reshape(N_NODES, 3, H)
    return (dx, dvec)


NC = 2
NS = 16
NW = NC * NS
P = 2
ROWS = NP // (NW * P)
CHUNK = 3200
NCHUNK = N_EDGES // CHUNK
G = 8


def _sc_body(nodetab, edgetab, dst_hbm, src_hbm, zrows_hbm, y_hbm,
             acc, sel_dst, sel_src, sel_eid,
             dstb0, srcb0, dstb1, srcb1,
             node_v0, edge_v0, node_v1, edge_v1, kref,
             sn0, se0, sn1, se1, sc0, sc1):
    c = lax.axis_index("c")
    s = lax.axis_index("s")
    z16 = jnp.zeros((16,), jnp.int32)
    gbufs = ((node_v0, edge_v0, sn0, se0), (node_v1, edge_v1, sn1, se1))
    cbufs = ((dstb0, srcb0, sc0), (dstb1, srcb1, sc1))

    def chunk_refs(ch, cbuf):
        db, sb, sem = cbuf
        return ((dst_hbm.at[pl.ds(ch * CHUNK, CHUNK)], db, sem),
                (src_hbm.at[pl.ds(ch * CHUNK, CHUNK)], sb, sem))

    def chunk_start(ch, cbuf):
        for srcr, dstr, sem in chunk_refs(ch, cbuf):
            pltpu.async_copy(srcr, dstr, sem)

    def chunk_wait(ch, cbuf):
        for srcr, dstr, sem in chunk_refs(ch, cbuf):
            pltpu.make_async_copy(srcr, dstr, sem).wait()

    def gather_refs(b, buf):
        nv, ev_, s1, s2 = buf
        return ((nodetab.at[sel_src.at[pl.ds(b * G, G)]], nv, s1),
                (edgetab.at[sel_eid.at[pl.ds(b * G, G)]], ev_, s2))

    def gather_start(b, buf):
        for srcr, dstr, sem in gather_refs(b, buf):
            pltpu.async_copy(srcr, dstr, sem)

    def gather_wait(b, buf):
        for srcr, dstr, sem in gather_refs(b, buf):
            pltpu.make_async_copy(srcr, dstr, sem).wait()

    def run_filter(base, lo, db, sb):
        def filt(i, ptr):
            d16 = db[pl.ds(i * 16, 16)]
            dl = d16 - lo
            m = (dl >= 0) & (dl < ROWS)
            cnt = plsc.all_reduce_population_count(m)[0]

            def hit():
                s16 = sb[pl.ds(i * 16, 16)]
                pos = ptr + plsc.cumsum(m.astype(jnp.int32)) - 1
                plsc.store_scatter(sel_dst, [pos], dl, mask=m)
                plsc.store_scatter(sel_src, [pos], s16, mask=m)
                eid = base + i * 16 + lax.iota(jnp.int32, 16)
                plsc.store_scatter(sel_eid, [pos], eid, mask=m)
                return ptr + cnt

            return lax.cond(cnt > 0, hit, lambda: ptr)

        return lax.fori_loop(0, CHUNK // 16, filt, jnp.int32(0))

    def compute(b, buf):
        nv, ev_, _, _ = buf

        @plsc.parallel_loop(0, G, unroll=8)
        def body(g):
            g16 = jnp.full((16,), g, jnp.int32)
            drow = sel_dst[pl.ds(b * G + g, 16)][0]
            ev = [plsc.load_gather(
                ev_, [g16, jnp.full((16,), 3 * H + d, jnp.int32)])
                for d in range(3)]
            for k in range(H // 16):
                o = k * 16
                xh1 = nv[g, pl.ds(o, 16)]
                xh2 = nv[g, pl.ds(H + o, 16)]
                xh3 = nv[g, pl.ds(2 * H + o, 16)]
                r1 = ev_[g, pl.ds(o, 16)]
                r2 = ev_[g, pl.ds(H + o, 16)]
                r3 = ev_[g, pl.ds(2 * H + o, 16)]
                a = xh2 * r2
                bb = xh3 * r3
                plsc.addupdate(acc.at[drow, pl.ds(o, 16)], xh1 * r1)
                for d in range(3):
                    vd = nv[g, pl.ds(3 * H + d * H + o, 16)]
                    plsc.addupdate(acc.at[drow, pl.ds(H + d * H + o, 16)],
                                   vd * a + ev[d] * bb)

    def pass_body(p, _):
        w = p * NW + s * NC + c
        lo = w * ROWS
        pltpu.sync_copy(zrows_hbm, acc)
        chunk_start(0, cbufs[0])

        def chunk_body(ch, _):
            base = ch * CHUNK
            for cpar in range(2):
                @pl.when(ch % 2 == cpar)
                def _():
                    @pl.when(ch + 1 < NCHUNK)
                    def _():
                        chunk_start(ch + 1, cbufs[1 - cpar])
                    chunk_wait(ch, cbufs[cpar])
                    kref[0] = run_filter(base, lo, cbufs[cpar][0],
                                         cbufs[cpar][1])
            K = kref[0]
            sel_dst[pl.ds(K, 16)] = z16
            sel_src[pl.ds(K, 16)] = z16 + N_NODES
            sel_eid[pl.ds(K, 16)] = z16
            nb = (K + G - 1) // G

            @pl.when(nb > 0)
            def _():
                gather_start(0, gbufs[0])

            def batch(b, _):
                for par in range(2):
                    @pl.when(b % 2 == par)
                    def _():
                        @pl.when(b + 1 < nb)
                        def _():
                            gather_start(b + 1, gbufs[1 - par])
                        gather_wait(b, gbufs[par])
                        compute(b, gbufs[par])
                return 0

            lax.fori_loop(0, nb, batch, 0)
            return 0

        lax.fori_loop(0, NCHUNK, chunk_body, 0)
        pltpu.sync_copy(acc, y_hbm.at[pl.ds(lo, ROWS)])
        return 0

    lax.fori_loop(0, P, pass_body, 0)


def _sc_edge_kernel():
    return pl.kernel(
        _sc_body,
        out_type=jax.ShapeDtypeStruct((NP, 512), jnp.float32),
        mesh=plsc.VectorSubcoreMesh(core_axis_name="c", subcore_axis_name="s"),
        scratch_types=[
            pltpu.VMEM((ROWS, 512), jnp.float32),
            pltpu.VMEM((CHUNK + 32,), jnp.int32),
            pltpu.VMEM((CHUNK + 32,), jnp.int32),
            pltpu.VMEM((CHUNK + 32,), jnp.int32),
            pltpu.VMEM((CHUNK,), jnp.int32),
            pltpu.VMEM((CHUNK,), jnp.int32),
            pltpu.VMEM((CHUNK,), jnp.int32),
            pltpu.VMEM((CHUNK,), jnp.int32),
            pltpu.VMEM((G, 6 * H), jnp.float32),
            pltpu.VMEM((G, 512), jnp.float32),
            pltpu.VMEM((G, 6 * H), jnp.float32),
            pltpu.VMEM((G, 512), jnp.float32),
            pltpu.SMEM((1,), jnp.int32),
            pltpu.SemaphoreType.DMA,
            pltpu.SemaphoreType.DMA,
            pltpu.SemaphoreType.DMA,
            pltpu.SemaphoreType.DMA,
            pltpu.SemaphoreType.DMA,
            pltpu.SemaphoreType.DMA,
        ],
        compiler_params=pltpu.CompilerParams(needs_layout_passes=False),
    )


# speedup vs baseline: 1.6994x; 1.6994x over previous
import functools
import math

import jax
import jax.numpy as jnp
from jax import lax
from jax.experimental import pallas as pl
from jax.experimental.pallas import tpu as pltpu
from jax.experimental.pallas import tpu_sc as plsc

H = 128
NUM_RBF = 32
N_NODES = 10000
N_EDGES = 320000

NP = 10240


def _node_body(x_ref, vecf_ref, w1_ref, b1_ref, w2_ref, b2_ref, g_ref, b_ref,
               out_ref, *, block_rows):
    x = x_ref[...]
    mu = jnp.mean(x, axis=-1, keepdims=True)
    var = jnp.mean((x - mu) ** 2, axis=-1, keepdims=True)
    xn = (x - mu) * lax.rsqrt(var + 1e-5) * g_ref[...] + b_ref[...]
    t = jnp.dot(xn, w1_ref[...], preferred_element_type=jnp.float32) + b1_ref[...]
    t = jax.nn.silu(t) * (1.0 / 0.6)
    xh = jnp.dot(t, w2_ref[...], preferred_element_type=jnp.float32) + b2_ref[...]
    row = pl.program_id(0) * block_rows + lax.broadcasted_iota(
        jnp.int32, (block_rows, 1), 0)
    xh = jnp.where(row < N_NODES, xh, 0.0)
    out_ref[:, :3 * H] = xh
    out_ref[:, 3 * H:] = jnp.where(row < N_NODES, vecf_ref[...], 0.0)


def _make_nodetab(xp, vecfp):
    blk = 1024
    grid = NP // blk
    return pl.pallas_call(
        functools.partial(_node_body, block_rows=blk),
        grid=(grid,),
        in_specs=[
            pl.BlockSpec((blk, H), lambda i: (i, 0)),
            pl.BlockSpec((blk, 3 * H), lambda i: (i, 0)),
            pl.BlockSpec((H, H), lambda i: (0, 0)),
            pl.BlockSpec((H,), lambda i: (0,)),
            pl.BlockSpec((H, 3 * H), lambda i: (0, 0)),
            pl.BlockSpec((3 * H,), lambda i: (0,)),
            pl.BlockSpec((H,), lambda i: (0,)),
            pl.BlockSpec((H,), lambda i: (0,)),
        ],
        out_specs=pl.BlockSpec((blk, 6 * H), lambda i: (i, 0)),
        out_shape=jax.ShapeDtypeStruct((NP, 6 * H), jnp.float32),
    )


def _edge_body(rbf_ref, w_ref, b_ref, evp_ref, out_ref):
    blk = rbf_ref.shape[0]
    out_ref[:, :3 * H] = jnp.dot(rbf_ref[...], w_ref[...],
                                 preferred_element_type=jnp.float32) + b_ref[...]
    out_ref[:, 3 * H:3 * H + 16] = evp_ref[...]
    out_ref[:, 3 * H + 16:] = jnp.zeros((blk, 512 - 3 * H - 16), jnp.float32)


def _make_edgetab():
    blk = 3200
    grid = N_EDGES // blk
    return pl.pallas_call(
        _edge_body,
        grid=(grid,),
        in_specs=[
            pl.BlockSpec((blk, NUM_RBF), lambda i: (i, 0)),
            pl.BlockSpec((NUM_RBF, 3 * H), lambda i: (0, 0)),
            pl.BlockSpec((3 * H,), lambda i: (0,)),
            pl.BlockSpec((blk, 16), lambda i: (i, 0)),
        ],
        out_specs=pl.BlockSpec((blk, 512), lambda i: (i, 0)),
        out_shape=jax.ShapeDtypeStruct((N_EDGES, 512), jnp.float32),
    )


def kernel(x, vec, edge_index, edge_rbf, edge_vector, W1, b1, W2, b2, Wrbf,
           brbf, ln_g, ln_b):
    inv3 = 1.0 / math.sqrt(3.0)
    invh = 1.0 / math.sqrt(float(H))
    s = jnp.concatenate([
        jnp.full((H,), 1.0, jnp.float32),
        jnp.full((H,), inv3 * invh, jnp.float32),
        jnp.full((H,), invh, jnp.float32),
    ])
    Wrbf_s = Wrbf * s[None, :]
    brbf_s = brbf * s

    xp = jnp.pad(x, ((0, NP - N_NODES), (0, 0)))
    vecfp = jnp.pad(vec.reshape(N_NODES, 3 * H), ((0, NP - N_NODES), (0, 0)))
    evp = jnp.pad(edge_vector, ((0, 0), (0, 13)))

    nodetab = _make_nodetab(xp, vecfp)(
        xp, vecfp, W1, b1, W2, b2, ln_g, ln_b)
    edgetab = _make_edgetab()(edge_rbf, Wrbf_s, brbf_s, evp)

    src = edge_index[0].astype(jnp.int32)
    dst = edge_index[1].astype(jnp.int32)

    zrows = jnp.zeros((ROWS, 512), jnp.float32)
    Y = _sc_edge_kernel()(nodetab, edgetab, dst, src, zrows)

    dx = Y[:N_NODES, :H]
    dvec = Y[:N_NODES, H:].reshape(N_NODES, 3, H)
    return (dx, dvec)


NC = 2
NS = 16
NW = NC * NS
P = 2
ROWS = NP // (NW * P)
CHUNK = 3200
NCHUNK = N_EDGES // CHUNK
G = 8


def _sc_body(nodetab, edgetab, dst_hbm, src_hbm, zrows_hbm, y_hbm,
             acc, sel_dst, sel_src, sel_eid,
             dstb0, srcb0, dstb1, srcb1,
             node_v0, edge_v0, node_v1, edge_v1, kref,
             sn0, se0, sn1, se1, sc0, sc1):
    c = lax.axis_index("c")
    s = lax.axis_index("s")
    z16 = jnp.zeros((16,), jnp.int32)
    gbufs = ((node_v0, edge_v0, sn0, se0), (node_v1, edge_v1, sn1, se1))
    cbufs = ((dstb0, srcb0, sc0), (dstb1, srcb1, sc1))

    def chunk_refs(ch, cbuf):
        db, sb, sem = cbuf
        return ((dst_hbm.at[pl.ds(ch * CHUNK, CHUNK)], db, sem),
                (src_hbm.at[pl.ds(ch * CHUNK, CHUNK)], sb, sem))

    def chunk_start(ch, cbuf):
        for srcr, dstr, sem in chunk_refs(ch, cbuf):
            pltpu.async_copy(srcr, dstr, sem)

    def chunk_wait(ch, cbuf):
        for srcr, dstr, sem in chunk_refs(ch, cbuf):
            pltpu.make_async_copy(srcr, dstr, sem).wait()

    def gather_refs(b, buf):
        nv, ev_, s1, s2 = buf
        return ((nodetab.at[sel_src.at[pl.ds(b * G, G)]], nv, s1),
                (edgetab.at[sel_eid.at[pl.ds(b * G, G)]], ev_, s2))

    def gather_start(b, buf):
        for srcr, dstr, sem in gather_refs(b, buf):
            pltpu.async_copy(srcr, dstr, sem)

    def gather_wait(b, buf):
        for srcr, dstr, sem in gather_refs(b, buf):
            pltpu.make_async_copy(srcr, dstr, sem).wait()

    def run_filter(base, lo, db, sb):
        def filt(i, ptr):
            ms = []
            total = jnp.int32(0)
            for j in range(4):
                dl = db[pl.ds(i * 64 + j * 16, 16)] - lo
                m = (dl >= 0) & (dl < ROWS)
                cnt = plsc.all_reduce_population_count(m)[0]
                ms.append((m, dl, cnt))
                total = total + cnt

            def hit():
                q = ptr
                for j, (m, dl, cnt) in enumerate(ms):
                    s16 = sb[pl.ds(i * 64 + j * 16, 16)]
                    pos = q + plsc.cumsum(m.astype(jnp.int32)) - 1
                    plsc.store_scatter(sel_dst, [pos], dl, mask=m)
                    plsc.store_scatter(sel_src, [pos], s16, mask=m)
                    eid = base + i * 64 + j * 16 + lax.iota(jnp.int32, 16)
                    plsc.store_scatter(sel_eid, [pos], eid, mask=m)
                    q = q + cnt
                return q

            return lax.cond(total > 0, hit, lambda: ptr)

        return lax.fori_loop(0, CHUNK // 64, filt, jnp.int32(0))

    def compute(b, buf):
        nv, ev_, _, _ = buf

        @plsc.parallel_loop(0, G, unroll=4)
        def body(g):
            g16 = jnp.full((16,), g, jnp.int32)
            drow = sel_dst[pl.ds(b * G + g, 16)][0]
            ev = [plsc.load_gather(
                ev_, [g16, jnp.full((16,), 3 * H + d, jnp.int32)])
                for d in range(3)]
            for k in range(H // 16):
                o = k * 16
                xh1 = nv[g, pl.ds(o, 16)]
                xh2 = nv[g, pl.ds(H + o, 16)]
                xh3 = nv[g, pl.ds(2 * H + o, 16)]
                r1 = ev_[g, pl.ds(o, 16)]
                r2 = ev_[g, pl.ds(H + o, 16)]
                r3 = ev_[g, pl.ds(2 * H + o, 16)]
                a = xh2 * r2
                bb = xh3 * r3
                plsc.addupdate(acc.at[drow, pl.ds(o, 16)], xh1 * r1)
                for d in range(3):
                    vd = nv[g, pl.ds(3 * H + d * H + o, 16)]
                    plsc.addupdate(acc.at[drow, pl.ds(H + d * H + o, 16)],
                                   vd * a + ev[d] * bb)

    def pass_body(p, _):
        w = p * NW + s * NC + c
        lo = w * ROWS
        pltpu.sync_copy(zrows_hbm, acc)
        chunk_start(0, cbufs[0])

        def chunk_body(ch, _):
            base = ch * CHUNK
            for cpar in range(2):
                @pl.when(ch % 2 == cpar)
                def _():
                    @pl.when(ch + 1 < NCHUNK)
                    def _():
                        chunk_start(ch + 1, cbufs[1 - cpar])
                    chunk_wait(ch, cbufs[cpar])
                    kref[0] = run_filter(base, lo, cbufs[cpar][0],
                                         cbufs[cpar][1])
            K = kref[0]
            sel_dst[pl.ds(K, 16)] = z16
            sel_src[pl.ds(K, 16)] = z16 + N_NODES
            sel_eid[pl.ds(K, 16)] = z16
            nb = (K + G - 1) // G

            @pl.when(nb > 0)
            def _():
                gather_start(0, gbufs[0])

            def batch(b, _):
                for par in range(2):
                    @pl.when(b % 2 == par)
                    def _():
                        @pl.when(b + 1 < nb)
                        def _():
                            gather_start(b + 1, gbufs[1 - par])
                        gather_wait(b, gbufs[par])
                        compute(b, gbufs[par])
                return 0

            lax.fori_loop(0, nb, batch, 0)
            return 0

        lax.fori_loop(0, NCHUNK, chunk_body, 0)
        pltpu.sync_copy(acc, y_hbm.at[pl.ds(lo, ROWS)])
        return 0

    lax.fori_loop(0, P, pass_body, 0)


def _sc_edge_kernel():
    return pl.kernel(
        _sc_body,
        out_type=jax.ShapeDtypeStruct((NP, 512), jnp.float32),
        mesh=plsc.VectorSubcoreMesh(core_axis_name="c", subcore_axis_name="s"),
        scratch_types=[
            pltpu.VMEM((ROWS, 512), jnp.float32),
            pltpu.VMEM((CHUNK + 32,), jnp.int32),
            pltpu.VMEM((CHUNK + 32,), jnp.int32),
            pltpu.VMEM((CHUNK + 32,), jnp.int32),
            pltpu.VMEM((CHUNK,), jnp.int32),
            pltpu.VMEM((CHUNK,), jnp.int32),
            pltpu.VMEM((CHUNK,), jnp.int32),
            pltpu.VMEM((CHUNK,), jnp.int32),
            pltpu.VMEM((G, 6 * H), jnp.float32),
            pltpu.VMEM((G, 512), jnp.float32),
            pltpu.VMEM((G, 6 * H), jnp.float32),
            pltpu.VMEM((G, 512), jnp.float32),
            pltpu.SMEM((1,), jnp.int32),
            pltpu.SemaphoreType.DMA,
            pltpu.SemaphoreType.DMA,
            pltpu.SemaphoreType.DMA,
            pltpu.SemaphoreType.DMA,
            pltpu.SemaphoreType.DMA,
            pltpu.SemaphoreType.DMA,
        ],
        compiler_params=pltpu.CompilerParams(needs_layout_passes=False),
    )


# speedup vs baseline: 1.7599x; 1.0356x over previous
import functools
import math

import jax
import jax.numpy as jnp
from jax import lax
from jax.experimental import pallas as pl
from jax.experimental.pallas import tpu as pltpu
from jax.experimental.pallas import tpu_sc as plsc

H = 128
NUM_RBF = 32
N_NODES = 10000
N_EDGES = 320000

NP = 10240


def _node_body(x_ref, vecf_ref, w1_ref, b1_ref, w2_ref, b2_ref, g_ref, b_ref,
               out_ref, *, block_rows):
    x = x_ref[...]
    mu = jnp.mean(x, axis=-1, keepdims=True)
    var = jnp.mean((x - mu) ** 2, axis=-1, keepdims=True)
    xn = (x - mu) * lax.rsqrt(var + 1e-5) * g_ref[...] + b_ref[...]
    t = jnp.dot(xn, w1_ref[...], preferred_element_type=jnp.float32) + b1_ref[...]
    t = jax.nn.silu(t) * (1.0 / 0.6)
    xh = jnp.dot(t, w2_ref[...], preferred_element_type=jnp.float32) + b2_ref[...]
    row = pl.program_id(0) * block_rows + lax.broadcasted_iota(
        jnp.int32, (block_rows, 1), 0)
    xh = jnp.where(row < N_NODES, xh, 0.0)
    out_ref[:, :3 * H] = xh
    out_ref[:, 3 * H:] = jnp.where(row < N_NODES, vecf_ref[...], 0.0)


def _make_nodetab(xp, vecfp):
    blk = 1024
    grid = NP // blk
    return pl.pallas_call(
        functools.partial(_node_body, block_rows=blk),
        grid=(grid,),
        in_specs=[
            pl.BlockSpec((blk, H), lambda i: (i, 0)),
            pl.BlockSpec((blk, 3 * H), lambda i: (i, 0)),
            pl.BlockSpec((H, H), lambda i: (0, 0)),
            pl.BlockSpec((H,), lambda i: (0,)),
            pl.BlockSpec((H, 3 * H), lambda i: (0, 0)),
            pl.BlockSpec((3 * H,), lambda i: (0,)),
            pl.BlockSpec((H,), lambda i: (0,)),
            pl.BlockSpec((H,), lambda i: (0,)),
        ],
        out_specs=pl.BlockSpec((blk, 6 * H), lambda i: (i, 0)),
        out_shape=jax.ShapeDtypeStruct((NP, 6 * H), jnp.float32),
    )


def _edge_body(rbf_ref, w_ref, b_ref, evp_ref, out_ref):
    blk = rbf_ref.shape[0]
    out_ref[:, :3 * H] = jnp.dot(rbf_ref[...], w_ref[...],
                                 preferred_element_type=jnp.float32) + b_ref[...]
    out_ref[:, 3 * H:3 * H + 16] = evp_ref[...]
    out_ref[:, 3 * H + 16:] = jnp.zeros((blk, 512 - 3 * H - 16), jnp.float32)


def _make_edgetab():
    blk = 3200
    grid = N_EDGES // blk
    return pl.pallas_call(
        _edge_body,
        grid=(grid,),
        in_specs=[
            pl.BlockSpec((blk, NUM_RBF), lambda i: (i, 0)),
            pl.BlockSpec((NUM_RBF, 3 * H), lambda i: (0, 0)),
            pl.BlockSpec((3 * H,), lambda i: (0,)),
            pl.BlockSpec((blk, 16), lambda i: (i, 0)),
        ],
        out_specs=pl.BlockSpec((blk, 512), lambda i: (i, 0)),
        out_shape=jax.ShapeDtypeStruct((N_EDGES, 512), jnp.float32),
    )


def kernel(x, vec, edge_index, edge_rbf, edge_vector, W1, b1, W2, b2, Wrbf,
           brbf, ln_g, ln_b):
    inv3 = 1.0 / math.sqrt(3.0)
    invh = 1.0 / math.sqrt(float(H))
    s = jnp.concatenate([
        jnp.full((H,), 1.0, jnp.float32),
        jnp.full((H,), inv3 * invh, jnp.float32),
        jnp.full((H,), invh, jnp.float32),
    ])
    Wrbf_s = Wrbf * s[None, :]
    brbf_s = brbf * s

    xp = jnp.pad(x, ((0, NP - N_NODES), (0, 0)))
    vecfp = jnp.pad(vec.reshape(N_NODES, 3 * H), ((0, NP - N_NODES), (0, 0)))
    evp = jnp.pad(edge_vector, ((0, 0), (0, 13)))

    nodetab = _make_nodetab(xp, vecfp)(
        xp, vecfp, W1, b1, W2, b2, ln_g, ln_b)
    edgetab = _make_edgetab()(edge_rbf, Wrbf_s, brbf_s, evp)

    src = edge_index[0].astype(jnp.int32)
    dst = edge_index[1].astype(jnp.int32)

    zrows = jnp.zeros((ROWS, 512), jnp.float32)
    Y = _sc_edge_kernel()(nodetab, edgetab, dst, src, zrows)

    dx = Y[:N_NODES, :H]
    dvec = Y[:N_NODES, H:].reshape(N_NODES, 3, H)
    return (dx, dvec)


NC = 2
NS = 16
NW = NC * NS
P = 2
ROWS = NP // (NW * P)
CHUNK = 3200
NCHUNK = N_EDGES // CHUNK
G = 8


def _sc_body(nodetab, edgetab, dst_hbm, src_hbm, zrows_hbm, y_hbm,
             acc, sel_dst, sel_src, sel_eid,
             dstb0, srcb0, dstb1, srcb1,
             node_v0, edge_v0, node_v1, edge_v1, kref,
             sn0, se0, sn1, se1, sc0, sc1):
    c = lax.axis_index("c")
    s = lax.axis_index("s")
    z16 = jnp.zeros((16,), jnp.int32)
    gbufs = ((node_v0, edge_v0, sn0, se0), (node_v1, edge_v1, sn1, se1))
    cbufs = ((dstb0, srcb0, sc0), (dstb1, srcb1, sc1))

    def chunk_refs(ch, cbuf):
        db, sb, sem = cbuf
        return ((dst_hbm.at[pl.ds(ch * CHUNK, CHUNK)], db, sem),
                (src_hbm.at[pl.ds(ch * CHUNK, CHUNK)], sb, sem))

    def chunk_start(ch, cbuf):
        for srcr, dstr, sem in chunk_refs(ch, cbuf):
            pltpu.async_copy(srcr, dstr, sem)

    def chunk_wait(ch, cbuf):
        for srcr, dstr, sem in chunk_refs(ch, cbuf):
            pltpu.make_async_copy(srcr, dstr, sem).wait()

    def gather_refs(b, buf):
        nv, ev_, s1, s2 = buf
        return ((nodetab.at[sel_src.at[pl.ds(b * G, G)]], nv, s1),
                (edgetab.at[sel_eid.at[pl.ds(b * G, G)]], ev_, s2))

    def gather_start(b, buf):
        for srcr, dstr, sem in gather_refs(b, buf):
            pltpu.async_copy(srcr, dstr, sem)

    def gather_wait(b, buf):
        for srcr, dstr, sem in gather_refs(b, buf):
            pltpu.make_async_copy(srcr, dstr, sem).wait()

    def run_filter(base, lo, db, sb):
        def filt(i, ptr):
            ms = []
            total = jnp.int32(0)
            for j in range(8):
                dl = db[pl.ds(i * 128 + j * 16, 16)] - lo
                m = (dl >= 0) & (dl < ROWS)
                cnt = plsc.all_reduce_population_count(m)[0]
                ms.append((m, dl, cnt))
                total = total + cnt

            def hit():
                q = ptr
                for j, (m, dl, cnt) in enumerate(ms):
                    s16 = sb[pl.ds(i * 128 + j * 16, 16)]
                    pos = q + plsc.cumsum(m.astype(jnp.int32)) - 1
                    plsc.store_scatter(sel_dst, [pos], dl, mask=m)
                    plsc.store_scatter(sel_src, [pos], s16, mask=m)
                    eid = base + i * 128 + j * 16 + lax.iota(jnp.int32, 16)
                    plsc.store_scatter(sel_eid, [pos], eid, mask=m)
                    q = q + cnt
                return q

            return lax.cond(total > 0, hit, lambda: ptr)

        return lax.fori_loop(0, CHUNK // 128, filt, jnp.int32(0))

    def compute(b, buf):
        nv, ev_, _, _ = buf

        @plsc.parallel_loop(0, G, unroll=4)
        def body(g):
            g16 = jnp.full((16,), g, jnp.int32)
            drow = sel_dst[pl.ds(b * G + g, 16)][0]
            ev = [plsc.load_gather(
                ev_, [g16, jnp.full((16,), 3 * H + d, jnp.int32)])
                for d in range(3)]
            for k in range(H // 16):
                o = k * 16
                xh1 = nv[g, pl.ds(o, 16)]
                xh2 = nv[g, pl.ds(H + o, 16)]
                xh3 = nv[g, pl.ds(2 * H + o, 16)]
                r1 = ev_[g, pl.ds(o, 16)]
                r2 = ev_[g, pl.ds(H + o, 16)]
                r3 = ev_[g, pl.ds(2 * H + o, 16)]
                a = xh2 * r2
                bb = xh3 * r3
                plsc.addupdate(acc.at[drow, pl.ds(o, 16)], xh1 * r1)
                for d in range(3):
                    vd = nv[g, pl.ds(3 * H + d * H + o, 16)]
                    plsc.addupdate(acc.at[drow, pl.ds(H + d * H + o, 16)],
                                   vd * a + ev[d] * bb)

    def pass_body(p, _):
        w = p * NW + s * NC + c
        lo = w * ROWS
        pltpu.sync_copy(zrows_hbm, acc)
        chunk_start(0, cbufs[0])

        def chunk_body(ch, _):
            base = ch * CHUNK
            for cpar in range(2):
                @pl.when(ch % 2 == cpar)
                def _():
                    @pl.when(ch + 1 < NCHUNK)
                    def _():
                        chunk_start(ch + 1, cbufs[1 - cpar])
                    chunk_wait(ch, cbufs[cpar])
                    kref[0] = run_filter(base, lo, cbufs[cpar][0],
                                         cbufs[cpar][1])
            K = kref[0]
            sel_dst[pl.ds(K, 16)] = z16
            sel_src[pl.ds(K, 16)] = z16 + N_NODES
            sel_eid[pl.ds(K, 16)] = z16
            nb = (K + G - 1) // G

            @pl.when(nb > 0)
            def _():
                gather_start(0, gbufs[0])

            def batch(b, _):
                for par in range(2):
                    @pl.when(b % 2 == par)
                    def _():
                        @pl.when(b + 1 < nb)
                        def _():
                            gather_start(b + 1, gbufs[1 - par])
                        gather_wait(b, gbufs[par])
                        compute(b, gbufs[par])
                return 0

            lax.fori_loop(0, nb, batch, 0)
            return 0

        lax.fori_loop(0, NCHUNK, chunk_body, 0)
        pltpu.sync_copy(acc, y_hbm.at[pl.ds(lo, ROWS)])
        return 0

    lax.fori_loop(0, P, pass_body, 0)


def _sc_edge_kernel():
    return pl.kernel(
        _sc_body,
        out_type=jax.ShapeDtypeStruct((NP, 512), jnp.float32),
        mesh=plsc.VectorSubcoreMesh(core_axis_name="c", subcore_axis_name="s"),
        scratch_types=[
            pltpu.VMEM((ROWS, 512), jnp.float32),
            pltpu.VMEM((CHUNK + 32,), jnp.int32),
            pltpu.VMEM((CHUNK + 32,), jnp.int32),
            pltpu.VMEM((CHUNK + 32,), jnp.int32),
            pltpu.VMEM((CHUNK,), jnp.int32),
            pltpu.VMEM((CHUNK,), jnp.int32),
            pltpu.VMEM((CHUNK,), jnp.int32),
            pltpu.VMEM((CHUNK,), jnp.int32),
            pltpu.VMEM((G, 6 * H), jnp.float32),
            pltpu.VMEM((G, 512), jnp.float32),
            pltpu.VMEM((G, 6 * H), jnp.float32),
            pltpu.VMEM((G, 512), jnp.float32),
            pltpu.SMEM((1,), jnp.int32),
            pltpu.SemaphoreType.DMA,
            pltpu.SemaphoreType.DMA,
            pltpu.SemaphoreType.DMA,
            pltpu.SemaphoreType.DMA,
            pltpu.SemaphoreType.DMA,
            pltpu.SemaphoreType.DMA,
        ],
        compiler_params=pltpu.CompilerParams(needs_layout_passes=False),
    )
